# baseline (speedup 1.0000x reference)
import jax
import jax.numpy as jnp
from jax import lax
from jax.experimental import pallas as pl

EMBED_DIM = 32
NUM_EMB = 24
PACK = 8
KDIM = PACK * NUM_EMB
MDIM = PACK * EMBED_DIM
BBLK = 16384


def kernel(batch, table):
    n_rows, seq = batch.shape
    batch_t = batch.T

    bd_t = jnp.einsum(
        "jJ,tc->jcJt", jnp.eye(PACK, dtype=table.dtype), table
    ).reshape(MDIM, KDIM)
    bd_hi = bd_t.astype(jnp.bfloat16)
    bd_lo = (bd_t - bd_hi.astype(jnp.float32)).astype(jnp.bfloat16)

    def body(idx_ref, bd_ref, lo_ref, out_ref):
        idx3 = jnp.broadcast_to(idx_ref[...][:, None, :], (PACK, NUM_EMB, BBLK))
        val3 = lax.broadcasted_iota(jnp.int32, (PACK, NUM_EMB, BBLK), 1)
        oh = (idx3 == val3).reshape(KDIM, BBLK).astype(jnp.bfloat16)
        res = jnp.dot(bd_ref[...], oh, preferred_element_type=jnp.float32)
        res += jnp.dot(lo_ref[...], oh, preferred_element_type=jnp.float32)
        out_ref[...] = res.reshape(PACK, EMBED_DIM, BBLK)

    out_t = pl.pallas_call(
        body,
        grid=(seq // PACK,),
        in_specs=[
            pl.BlockSpec((PACK, BBLK), lambda i: (i, 0)),
            pl.BlockSpec((MDIM, KDIM), lambda i: (0, 0)),
            pl.BlockSpec((MDIM, KDIM), lambda i: (0, 0)),
        ],
        out_specs=pl.BlockSpec((PACK, EMBED_DIM, BBLK), lambda i: (i, 0, 0)),
        out_shape=jax.ShapeDtypeStruct((seq, EMBED_DIM, n_rows), jnp.float32),
    )(batch_t, bd_hi, bd_lo)

    return jnp.transpose(out_t, (2, 0, 1))
